# baseline (speedup 1.0000x reference)
import functools

import jax
import jax.numpy as jnp
from jax import lax
from jax.experimental import pallas as pl
from jax.experimental.pallas import tpu as pltpu
from jax.experimental.pallas import tpu_sc as plsc

VOCAB = 256
B = 4096
L = 50

NC = 2
NS = 16
NW = NC * NS

TOTAL = B * L
BLK = B // NW
SUB = 64
NBUF = 6
H = BLK // SUB
NCH = L * H


def _sc_gather(idx_t, W):
    mesh = plsc.VectorSubcoreMesh(core_axis_name="c", subcore_axis_name="s")

    @functools.partial(
        pl.kernel,
        mesh=mesh,
        out_type=jax.ShapeDtypeStruct((TOTAL, VOCAB), jnp.float32),
        scratch_types=(
            [pltpu.VMEM((L * BLK,), jnp.int32)]
            + [pltpu.VMEM((SUB,), jnp.int32)] * NBUF
            + [pltpu.VMEM((SUB, VOCAB), jnp.float32)] * NBUF
            + [pltpu.SemaphoreType.DMA] * (2 * NBUF)
        ),
    )
    def k(idxt_hbm, w_hbm, out_hbm, idx_ws, *bufs):
        tri = list(bufs[:NBUF])
        rows = list(bufs[NBUF:2 * NBUF])
        gsem = list(bufs[2 * NBUF:3 * NBUF])
        osem = list(bufs[3 * NBUF:4 * NBUF])

        wid = lax.axis_index("s") * NC + lax.axis_index("c")
        b0 = wid * BLK
        pltpu.sync_copy(idxt_hbm.at[pl.ds(wid * (L * BLK), L * BLK)], idx_ws)

        def compute_tri(g, b):
            l = g // H
            h = g % H
            lm1 = jnp.maximum(l - 1, 0)
            valid = jnp.where(l == 0, 0, 1)
            for t in range(SUB // 16):
                cur = idx_ws[pl.ds(l * BLK + h * SUB + t * 16, 16)]
                prv = idx_ws[pl.ds(lm1 * BLK + h * SUB + t * 16, 16)]
                tri[b][pl.ds(t * 16, 16)] = (prv * VOCAB + cur) * valid

        def rbase(g):
            return (g // H) * B + b0 + (g % H) * SUB

        def start_gather(b):
            pltpu.async_copy(w_hbm.at[tri[b]], rows[b], gsem[b])

        def wait_gather(b):
            pltpu.make_async_copy(w_hbm.at[tri[b]], rows[b], gsem[b]).wait()

        def start_out(g, b):
            pltpu.async_copy(rows[b], out_hbm.at[pl.ds(rbase(g), SUB)], osem[b])

        def wait_out(g, b):
            pltpu.make_async_copy(rows[b], out_hbm.at[pl.ds(rbase(g), SUB)],
                                  osem[b]).wait()

        for g in range(NBUF):
            compute_tri(g, g)
            start_gather(g)
        wait_gather(0)
        start_out(0, 0)

        def step(j, fb):
            wait_out(j - 1, fb)
            compute_tri(j + NBUF - 1, fb)
            start_gather(fb)
            cb = (fb + 1) % NBUF
            wait_gather(cb)
            start_out(j, cb)

        full = NCH - NBUF
        ngroups = full // NBUF

        def group_body(i, carry):
            for jj in range(NBUF):
                step(i * NBUF + 1 + jj, jj)
            return carry

        lax.fori_loop(0, ngroups, group_body, 0)
        for kk in range(full % NBUF):
            j = ngroups * NBUF + 1 + kk
            step(j, (j - 1) % NBUF)

        for j in range(NCH - NBUF + 1, NCH):
            wait_out(j - 1, (j - 1) % NBUF)
            wait_gather(j % NBUF)
            start_out(j, j % NBUF)
        wait_out(NCH - 1, (NCH - 1) % NBUF)

    return k(idx_t, W)


def kernel(idx, W):
    idx_a = jnp.transpose(idx.astype(jnp.int32).reshape(NW, BLK, L),
                          (0, 2, 1)).reshape(NW * L * BLK)
    out = _sc_gather(idx_a, W)
    return jnp.transpose(out.reshape(L, B, VOCAB), (1, 0, 2))
